# baseline (speedup 1.0000x reference)
import functools

import numpy as np

import jax
import jax.numpy as jnp
from jax.experimental import pallas as pl
from jax.experimental.pallas import tpu as pltpu

_B = 1280
_BC = 640
_TR = 8
_TC = 16
_M = _B // _BC
_K = 1
_W = 4
_NSLOT = _M + 2 * _M


def _build_schedule():
    steps = []
    for r in range(_TR):
        for c in range(_TC):
            steps.append((r, c, 0, 1))
    for r in range(_TR):
        for c in range(_M * (r + _K + 1), _TC):
            steps.append((r, c, 1, 1))
    while len(steps) % _W:
        steps.append((0, 0, 1, 0))

    while True:
        n2 = len(steps) // _W

        last = {}
        for r in range(_TR):
            end_strip = min(r + _K, _TR - 1)
            last[r] = (end_strip * _TC + (_TC - 1)) // _W
        for t, (r, c, p2, live) in enumerate(steps):
            if p2 and live:
                last[r] = max(last[r], t // _W)

        order = sorted(range(_TR), key=lambda r: (last[r], r))
        copy_step = {}
        prev = -1
        ok = True
        for r in order:
            s = max(last[r], prev + 1)
            if s >= n2:
                ok = False
                break
            copy_step[r] = s
            prev = s
        if ok:
            break
        steps.extend([(0, 0, 1, 0)] * _W)

    out_idx = np.zeros(n2, np.int32)
    cpy = np.zeros(n2, np.int32)
    t0 = 0
    for r in order:
        out_idx[t0 : copy_step[r] + 1] = r
        cpy[copy_step[r]] = 1
        t0 = copy_step[r] + 1
    assert t0 == n2

    def arr(i):
        return np.array([s[i] for s in steps], np.int32).reshape(n2, _W).T.copy()

    return n2, arr(0), arr(1), arr(2), arr(3), out_idx, cpy


_NSTEPS, _R_ARR, _C_ARR, _P2_ARR, _LIVE_ARR, _OUT_IDX, _CPY = _build_schedule()


def _s1_body(x_ref, w1_ref, o_ref):
    o_ref[...] = jnp.dot(
        x_ref[...].astype(jnp.bfloat16),
        w1_ref[...].astype(jnp.bfloat16),
        preferred_element_type=jnp.float32,
    ).astype(jnp.bfloat16)


def _gcn_body(
    n_valid, nf,
    r_ref, c_ref, p2_ref, lv_ref, oi_ref, cp_ref,
    adj0_ref, adj1_ref, adj2_ref, adj3_ref,
    s10_ref, s11_ref, s12_ref, s13_ref,
    b1_ref, w2_ref, b2_ref,
    out_ref,
    h_acc, s1g, out_acc, held,
):
    t = pl.program_id(0)

    @pl.when(t == 0)
    def _init():
        s1g[...] = jnp.zeros_like(s1g)
        out_acc[...] = jnp.zeros_like(out_acc)

    def process(i, adj_ref, s1_ref):
        r = r_ref[i, t]
        c = c_ref[i, t]
        p2 = p2_ref[i, t]
        live = lv_ref[i, t]
        d = c // _M

        @pl.when((p2 == 0) & (r == 0))
        def _fill_s1():
            s1g[pl.ds(c * _BC, _BC), 0:nf] = s1_ref[...]

        def use_block(a):
            rhs = s1g[pl.ds(c * _BC, _BC), :]
            res = jnp.dot(a, rhs, preferred_element_type=jnp.float32)

            @pl.when(p2 == 0)
            def _layer1():
                @pl.when(c == 0)
                def _():
                    h_acc[...] = res[:, 0:nf]

                @pl.when(c != 0)
                def _():
                    h_acc[...] += res[:, 0:nf]

                @pl.when(d == r)
                def _():
                    held[jax.lax.rem(c, _M)] = a

                @pl.when(d == r + 1)
                def _():
                    held[_M + jax.lax.rem(c, 2 * _M)] = a

            @pl.when(((d < r) | (p2 == 1)) & (live == 1))
            def _layer2():
                out_acc[pl.ds(r * _B, _B), :] += res[:, nf:]

        a_raw = adj_ref[...].astype(jnp.bfloat16)

        @pl.when(c != _TC - 1)
        def _interior():
            use_block(a_raw)

        @pl.when(c == _TC - 1)
        def _edge():
            lane = jax.lax.broadcasted_iota(jnp.int32, (1, _BC), 1)
            use_block(jnp.where(lane < (n_valid - c * _BC), a_raw, 0))

        @pl.when((p2 == 0) & (c == _TC - 1))
        def _strip_end():
            h = jnp.maximum(h_acc[...] + b1_ref[...], 0.0)
            g_r = jnp.dot(
                h.astype(jnp.bfloat16), w2_ref[...],
                preferred_element_type=jnp.float32,
            )
            rows = jax.lax.broadcasted_iota(jnp.int32, g_r.shape, 0) + r * _B
            g_r = jnp.where(rows < n_valid, g_r, 0.0).astype(jnp.bfloat16)
            s1g[pl.ds(r * _B, _B), nf:] = g_r

            for j in range(_M):
                g_sub = g_r[j * _BC : (j + 1) * _BC, :]
                out_acc[pl.ds(r * _B, _B), :] += jnp.dot(
                    held[j], g_sub, preferred_element_type=jnp.float32
                )
                @pl.when(r >= 1)
                def _(j=j, g_sub=g_sub):
                    slot = _M + jax.lax.rem(_M * r + j, 2 * _M)
                    out_acc[pl.ds((r - 1) * _B, _B), :] += jnp.dot(
                        held[slot], g_sub, preferred_element_type=jnp.float32
                    )

    process(0, adj0_ref, s10_ref)
    process(1, adj1_ref, s11_ref)
    process(2, adj2_ref, s12_ref)
    process(3, adj3_ref, s13_ref)

    @pl.when(cp_ref[t] == 1)
    def _copy_out():
        orow = oi_ref[t]
        out_ref[...] = out_acc[pl.ds(orow * _B, _B), :] + b2_ref[...]


def kernel(x, adj, W1, b1, W2, b2):
    n, nfeat = x.shape
    nhid = W1.shape[1]
    nout = W2.shape[1]
    npad = _TR * _B
    b1r = b1.reshape(1, nhid)
    b2r = b2.reshape(1, nout)

    xp = jnp.pad(x, ((0, npad - n), (0, 0)))
    s1p = pl.pallas_call(
        _s1_body,
        out_shape=jax.ShapeDtypeStruct((npad, nhid), jnp.bfloat16),
    )(xp, W1)

    adj_spec = lambda i: pl.BlockSpec(
        (_B, _BC), lambda t, rr, cc, pp, ll, oo, kk, i=i: (rr[i, t], cc[i, t])
    )
    s1_spec = lambda i: pl.BlockSpec(
        (_BC, nhid), lambda t, rr, cc, pp, ll, oo, kk, i=i: (cc[i, t], 0)
    )
    grid_spec = pltpu.PrefetchScalarGridSpec(
        num_scalar_prefetch=6,
        grid=(_NSTEPS,),
        in_specs=[adj_spec(0), adj_spec(1), adj_spec(2), adj_spec(3),
                  s1_spec(0), s1_spec(1), s1_spec(2), s1_spec(3),
                  pl.BlockSpec((1, nhid), lambda t, rr, cc, pp, ll, oo, kk: (0, 0)),
                  pl.BlockSpec((nhid, nout), lambda t, rr, cc, pp, ll, oo, kk: (0, 0)),
                  pl.BlockSpec((1, nout), lambda t, rr, cc, pp, ll, oo, kk: (0, 0))],
        out_specs=pl.BlockSpec(
            (_B, nout), lambda t, rr, cc, pp, ll, oo, kk: (oo[t], 0)
        ),
        scratch_shapes=[
            pltpu.VMEM((_B, nhid), jnp.float32),
            pltpu.VMEM((npad, nhid + nout), jnp.bfloat16),
            pltpu.VMEM((npad, nout), jnp.float32),
            pltpu.VMEM((_NSLOT, _B, _BC), jnp.bfloat16),
        ],
    )

    out = pl.pallas_call(
        functools.partial(_gcn_body, n, nhid),
        grid_spec=grid_spec,
        out_shape=jax.ShapeDtypeStruct((n, nout), jnp.float32),
        compiler_params=pltpu.CompilerParams(
            dimension_semantics=("arbitrary",),
            vmem_limit_bytes=64 * 1024 * 1024,
        ),
    )(
        jnp.asarray(_R_ARR),
        jnp.asarray(_C_ARR),
        jnp.asarray(_P2_ARR),
        jnp.asarray(_LIVE_ARR),
        jnp.asarray(_OUT_IDX),
        jnp.asarray(_CPY),
        adj, adj, adj, adj,
        s1p, s1p, s1p, s1p,
        b1r,
        W2.astype(jnp.bfloat16),
        b2r,
    )
    return out


# speedup vs baseline: 1.1381x; 1.1381x over previous
import functools

import numpy as np

import jax
import jax.numpy as jnp
from jax.experimental import pallas as pl
from jax.experimental.pallas import tpu as pltpu

_B = 1280
_T = 8
_K = 1
_NSLOT = sum(k + 1 for k in range(1, _K + 1))


def _slot_base(k):
    return (k - 1) * (k + 2) // 2


def _build_schedule():
    steps = []
    for r in range(_T):
        for c in range(_T):
            steps.append((r, c, 0, 1))
    for r in range(_T):
        for c in range(r + _K + 1, _T):
            steps.append((r, c, 1, 1))
    if len(steps) % 2:
        steps.append((0, 0, 1, 0))
    n2 = len(steps) // 2

    last = {}
    for r in range(_T):
        end_strip = min(r + _K, _T - 1)
        last[r] = (end_strip * _T + (_T - 1)) // 2
    for t, (r, c, p2, live) in enumerate(steps):
        if p2 and live:
            last[r] = max(last[r], t // 2)

    order = sorted(range(_T), key=lambda r: (last[r], r))
    copy_step = {}
    prev = -1
    for r in order:
        s = max(last[r], prev + 1)
        assert s < n2
        copy_step[r] = s
        prev = s

    out_idx = np.zeros(n2, np.int32)
    cpy = np.zeros(n2, np.int32)
    t0 = 0
    for r in order:
        out_idx[t0 : copy_step[r] + 1] = r
        cpy[copy_step[r]] = 1
        t0 = copy_step[r] + 1
    assert t0 == n2

    def arr(i):
        return np.array([s[i] for s in steps], np.int32).reshape(n2, 2).T.copy()

    return n2, arr(0), arr(1), arr(2), arr(3), out_idx, cpy


_NSTEPS, _R_ARR, _C_ARR, _P2_ARR, _LIVE_ARR, _OUT_IDX, _CPY = _build_schedule()


def _s1_body(x_ref, w1_ref, o_ref):
    o_ref[...] = jnp.dot(
        x_ref[...].astype(jnp.bfloat16),
        w1_ref[...].astype(jnp.bfloat16),
        preferred_element_type=jnp.float32,
    ).astype(jnp.bfloat16)


def _gcn_body(
    n_valid, nf,
    r_ref, c_ref, p2_ref, lv_ref, oi_ref, cp_ref,
    adj0_ref, adj1_ref, s10_ref, s11_ref, b1_ref, w2_ref, b2_ref,
    out_ref,
    h_acc, s1g, out_acc, diag, held,
):
    t = pl.program_id(0)

    @pl.when(t == 0)
    def _init():
        s1g[...] = jnp.zeros_like(s1g)
        out_acc[...] = jnp.zeros_like(out_acc)

    def process(i, adj_ref, s1_ref):
        r = r_ref[i, t]
        c = c_ref[i, t]
        p2 = p2_ref[i, t]
        live = lv_ref[i, t]

        @pl.when((p2 == 0) & (r == 0))
        def _fill_s1():
            s1g[pl.ds(c * _B, _B), 0:nf] = s1_ref[...]

        def use_block(a):
            rhs = s1g[pl.ds(c * _B, _B), :]
            res = jnp.dot(a, rhs, preferred_element_type=jnp.float32)

            @pl.when(p2 == 0)
            def _layer1():
                @pl.when(c == 0)
                def _():
                    h_acc[...] = res[:, 0:nf]

                @pl.when(c != 0)
                def _():
                    h_acc[...] += res[:, 0:nf]

                @pl.when(c == r)
                def _():
                    diag[...] = a

                if _K > 0:
                    @pl.when((c > r) & (c <= r + _K))
                    def _():
                        k = c - r
                        base = (k - 1) * (k + 2) // 2
                        slot = base + jax.lax.rem(r, k + 1)
                        held[slot] = a

            @pl.when(((c < r) | (p2 == 1)) & (live == 1))
            def _layer2():
                out_acc[pl.ds(r * _B, _B), :] += res[:, nf:]

        a_raw = adj_ref[...].astype(jnp.bfloat16)

        @pl.when(c != _T - 1)
        def _interior():
            use_block(a_raw)

        @pl.when(c == _T - 1)
        def _edge():
            lane = jax.lax.broadcasted_iota(jnp.int32, (1, _B), 1)
            use_block(jnp.where(lane < (n_valid - c * _B), a_raw, 0))

        @pl.when((p2 == 0) & (c == _T - 1))
        def _strip_end():
            h = jnp.maximum(h_acc[...] + b1_ref[...], 0.0)
            g_r = jnp.dot(
                h.astype(jnp.bfloat16), w2_ref[...],
                preferred_element_type=jnp.float32,
            )
            rows = jax.lax.broadcasted_iota(jnp.int32, g_r.shape, 0) + r * _B
            g_r = jnp.where(rows < n_valid, g_r, 0.0).astype(jnp.bfloat16)
            s1g[pl.ds(r * _B, _B), nf:] = g_r
            out_acc[pl.ds(r * _B, _B), :] += jnp.dot(
                diag[...], g_r, preferred_element_type=jnp.float32
            )
            for kk in range(1, _K + 1):
                @pl.when(r >= kk)
                def _(kk=kk):
                    r2 = r - kk
                    slot = _slot_base(kk) + jax.lax.rem(r2, kk + 1)
                    out_acc[pl.ds(r2 * _B, _B), :] += jnp.dot(
                        held[slot], g_r, preferred_element_type=jnp.float32
                    )

    process(0, adj0_ref, s10_ref)
    process(1, adj1_ref, s11_ref)

    @pl.when(cp_ref[t] == 1)
    def _copy_out():
        orow = oi_ref[t]
        out_ref[...] = out_acc[pl.ds(orow * _B, _B), :] + b2_ref[...]


def kernel(x, adj, W1, b1, W2, b2):
    n, nfeat = x.shape
    nhid = W1.shape[1]
    nout = W2.shape[1]
    npad = _T * _B
    b1r = b1.reshape(1, nhid)
    b2r = b2.reshape(1, nout)

    xp = jnp.pad(x, ((0, npad - n), (0, 0)))
    s1p = pl.pallas_call(
        _s1_body,
        out_shape=jax.ShapeDtypeStruct((npad, nhid), jnp.bfloat16),
    )(xp, W1)

    held_shape = (_NSLOT, _B, _B) if _K > 0 else (1, 8, 128)
    grid_spec = pltpu.PrefetchScalarGridSpec(
        num_scalar_prefetch=6,
        grid=(_NSTEPS,),
        in_specs=[
            pl.BlockSpec((_B, _B), lambda t, rr, cc, pp, ll, oo, kk: (rr[0, t], cc[0, t])),
            pl.BlockSpec((_B, _B), lambda t, rr, cc, pp, ll, oo, kk: (rr[1, t], cc[1, t])),
            pl.BlockSpec((_B, nhid), lambda t, rr, cc, pp, ll, oo, kk: (cc[0, t], 0)),
            pl.BlockSpec((_B, nhid), lambda t, rr, cc, pp, ll, oo, kk: (cc[1, t], 0)),
            pl.BlockSpec((1, nhid), lambda t, rr, cc, pp, ll, oo, kk: (0, 0)),
            pl.BlockSpec((nhid, nout), lambda t, rr, cc, pp, ll, oo, kk: (0, 0)),
            pl.BlockSpec((1, nout), lambda t, rr, cc, pp, ll, oo, kk: (0, 0)),
        ],
        out_specs=pl.BlockSpec(
            (_B, nout), lambda t, rr, cc, pp, ll, oo, kk: (oo[t], 0)
        ),
        scratch_shapes=[
            pltpu.VMEM((_B, nhid), jnp.float32),
            pltpu.VMEM((npad, nhid + nout), jnp.bfloat16),
            pltpu.VMEM((npad, nout), jnp.float32),
            pltpu.VMEM((_B, _B), jnp.bfloat16),
            pltpu.VMEM(held_shape, jnp.bfloat16),
        ],
    )

    out = pl.pallas_call(
        functools.partial(_gcn_body, n, nhid),
        grid_spec=grid_spec,
        out_shape=jax.ShapeDtypeStruct((n, nout), jnp.float32),
        compiler_params=pltpu.CompilerParams(
            dimension_semantics=("arbitrary",),
            vmem_limit_bytes=64 * 1024 * 1024,
        ),
    )(
        jnp.asarray(_R_ARR),
        jnp.asarray(_C_ARR),
        jnp.asarray(_P2_ARR),
        jnp.asarray(_LIVE_ARR),
        jnp.asarray(_OUT_IDX),
        jnp.asarray(_CPY),
        adj,
        adj,
        s1p,
        s1p,
        b1r,
        W2.astype(jnp.bfloat16),
        b2r,
    )
    return out


# speedup vs baseline: 1.1737x; 1.0313x over previous
import functools

import numpy as np

import jax
import jax.numpy as jnp
from jax.experimental import pallas as pl
from jax.experimental.pallas import tpu as pltpu

_B = 1280
_T = 8
_K = 1
_NSLOT = sum(k + 1 for k in range(1, _K + 1))


def _slot_base(k):
    return (k - 1) * (k + 2) // 2


def _build_schedule():
    steps = []
    for r in range(_T):
        for c in range(_T):
            steps.append((r, c, 0, 1))
    for r in range(_T):
        for c in range(r + _K + 1, _T):
            steps.append((r, c, 1, 1))
    if len(steps) % 2:
        steps.append((0, 0, 1, 0))
    n2 = len(steps) // 2

    last = {}
    for r in range(_T):
        end_strip = min(r + _K, _T - 1)
        last[r] = (end_strip * _T + (_T - 1)) // 2
    for t, (r, c, p2, live) in enumerate(steps):
        if p2 and live:
            last[r] = max(last[r], t // 2)

    order = sorted(range(_T), key=lambda r: (last[r], r))
    copy_step = {}
    prev = -1
    for r in order:
        s = max(last[r], prev + 1)
        assert s < n2
        copy_step[r] = s
        prev = s

    out_idx = np.zeros(n2, np.int32)
    cpy = np.zeros(n2, np.int32)
    t0 = 0
    for r in order:
        out_idx[t0 : copy_step[r] + 1] = r
        cpy[copy_step[r]] = 1
        t0 = copy_step[r] + 1
    assert t0 == n2

    def arr(i):
        return np.array([s[i] for s in steps], np.int32).reshape(n2, 2).T.copy()

    return n2, arr(0), arr(1), arr(2), arr(3), out_idx, cpy


_NSTEPS, _R_ARR, _C_ARR, _P2_ARR, _LIVE_ARR, _OUT_IDX, _CPY = _build_schedule()


def _gcn_body(
    n_valid, nf,
    r_ref, c_ref, p2_ref, lv_ref, oi_ref, cp_ref,
    adj0_ref, adj1_ref, x_ref, w1_ref, b1_ref, w2_ref, b2_ref,
    out_ref,
    h_acc, s1g, out_acc, diag, held,
):
    t = pl.program_id(0)

    @pl.when(t == 0)
    def _init():
        s1g[...] = jnp.zeros_like(s1g)
        out_acc[...] = jnp.zeros_like(out_acc)
        s1g[0:n_valid, 0:nf] = jnp.dot(
            x_ref[...], w1_ref[...], preferred_element_type=jnp.float32
        ).astype(jnp.bfloat16)

    def process(i, adj_ref):
        r = r_ref[i, t]
        c = c_ref[i, t]
        p2 = p2_ref[i, t]
        live = lv_ref[i, t]

        def use_block(a):
            rhs = s1g[pl.ds(c * _B, _B), :]
            res = jnp.dot(a, rhs, preferred_element_type=jnp.float32)

            @pl.when(p2 == 0)
            def _layer1():
                @pl.when(c == 0)
                def _():
                    h_acc[...] = res[:, 0:nf]

                @pl.when(c != 0)
                def _():
                    h_acc[...] += res[:, 0:nf]

                @pl.when(c == r)
                def _():
                    diag[...] = a

                if _K > 0:
                    @pl.when((c > r) & (c <= r + _K))
                    def _():
                        k = c - r
                        base = (k - 1) * (k + 2) // 2
                        slot = base + jax.lax.rem(r, k + 1)
                        held[slot] = a

            @pl.when(((c < r) | (p2 == 1)) & (live == 1))
            def _layer2():
                out_acc[pl.ds(r * _B, _B), :] += res[:, nf:]

        a_raw = adj_ref[...].astype(jnp.bfloat16)

        @pl.when(c != _T - 1)
        def _interior():
            use_block(a_raw)

        @pl.when(c == _T - 1)
        def _edge():
            lane = jax.lax.broadcasted_iota(jnp.int32, (1, _B), 1)
            use_block(jnp.where(lane < (n_valid - c * _B), a_raw, 0))

        @pl.when((p2 == 0) & (c == _T - 1))
        def _strip_end():
            h = jnp.maximum(h_acc[...] + b1_ref[...], 0.0)
            g_r = jnp.dot(
                h.astype(jnp.bfloat16), w2_ref[...],
                preferred_element_type=jnp.float32,
            )
            rows = jax.lax.broadcasted_iota(jnp.int32, g_r.shape, 0) + r * _B
            g_r = jnp.where(rows < n_valid, g_r, 0.0).astype(jnp.bfloat16)
            s1g[pl.ds(r * _B, _B), nf:] = g_r
            out_acc[pl.ds(r * _B, _B), :] += jnp.dot(
                diag[...], g_r, preferred_element_type=jnp.float32
            )
            for kk in range(1, _K + 1):
                @pl.when(r >= kk)
                def _(kk=kk):
                    r2 = r - kk
                    slot = _slot_base(kk) + jax.lax.rem(r2, kk + 1)
                    out_acc[pl.ds(r2 * _B, _B), :] += jnp.dot(
                        held[slot], g_r, preferred_element_type=jnp.float32
                    )

    process(0, adj0_ref)
    process(1, adj1_ref)

    @pl.when(cp_ref[t] == 1)
    def _copy_out():
        orow = oi_ref[t]
        out_ref[...] = out_acc[pl.ds(orow * _B, _B), :] + b2_ref[...]


def kernel(x, adj, W1, b1, W2, b2):
    n, nfeat = x.shape
    nhid = W1.shape[1]
    nout = W2.shape[1]
    npad = _T * _B
    b1r = b1.reshape(1, nhid)
    b2r = b2.reshape(1, nout)

    held_shape = (_NSLOT, _B, _B) if _K > 0 else (1, 8, 128)
    grid_spec = pltpu.PrefetchScalarGridSpec(
        num_scalar_prefetch=6,
        grid=(_NSTEPS,),
        in_specs=[
            pl.BlockSpec((_B, _B), lambda t, rr, cc, pp, ll, oo, kk: (rr[0, t], cc[0, t])),
            pl.BlockSpec((_B, _B), lambda t, rr, cc, pp, ll, oo, kk: (rr[1, t], cc[1, t])),
            pl.BlockSpec((n, nfeat), lambda t, rr, cc, pp, ll, oo, kk: (0, 0)),
            pl.BlockSpec((nfeat, nhid), lambda t, rr, cc, pp, ll, oo, kk: (0, 0)),
            pl.BlockSpec((1, nhid), lambda t, rr, cc, pp, ll, oo, kk: (0, 0)),
            pl.BlockSpec((nhid, nout), lambda t, rr, cc, pp, ll, oo, kk: (0, 0)),
            pl.BlockSpec((1, nout), lambda t, rr, cc, pp, ll, oo, kk: (0, 0)),
        ],
        out_specs=pl.BlockSpec(
            (_B, nout), lambda t, rr, cc, pp, ll, oo, kk: (oo[t], 0)
        ),
        scratch_shapes=[
            pltpu.VMEM((_B, nhid), jnp.float32),
            pltpu.VMEM((npad, nhid + nout), jnp.bfloat16),
            pltpu.VMEM((npad, nout), jnp.float32),
            pltpu.VMEM((_B, _B), jnp.bfloat16),
            pltpu.VMEM(held_shape, jnp.bfloat16),
        ],
    )

    out = pl.pallas_call(
        functools.partial(_gcn_body, n, nhid),
        grid_spec=grid_spec,
        out_shape=jax.ShapeDtypeStruct((n, nout), jnp.float32),
        compiler_params=pltpu.CompilerParams(
            dimension_semantics=("arbitrary",),
            vmem_limit_bytes=64 * 1024 * 1024,
        ),
    )(
        jnp.asarray(_R_ARR),
        jnp.asarray(_C_ARR),
        jnp.asarray(_P2_ARR),
        jnp.asarray(_LIVE_ARR),
        jnp.asarray(_OUT_IDX),
        jnp.asarray(_CPY),
        adj,
        adj,
        x.astype(jnp.bfloat16),
        W1.astype(jnp.bfloat16),
        b1r,
        W2.astype(jnp.bfloat16),
        b2r,
    )
    return out
